# baseline (speedup 1.0000x reference)
import jax
import jax.numpy as jnp
from jax import lax
from jax.experimental import pallas as pl
from jax.experimental.pallas import tpu as pltpu
from jax.experimental.pallas import tpu_sc as plsc

NC, NS, L = 2, 16, 16
NW = NC * NS
T = 1024 * 50
TPW = T // NW
C = 80
NCH = TPW // C
DW, DF = 100, 256
D = DW + DF
DWP = 128


def _sc_body(x_hbm, wt_hbm, ft_hbm, out_hbm, idx_v, word_v, flair_v, comb_v,
             sem_w, sem_f, sem_o):
    wid = lax.axis_index("s") * NC + lax.axis_index("c")
    pltpu.sync_copy(x_hbm.at[pl.ds(wid * TPW, TPW)], idx_v)

    lanes = lax.iota(jnp.int32, L)

    def start_gathers(j, p):
        idx_row = idx_v.at[pl.ds(j * C, C)]
        pltpu.async_copy(wt_hbm.at[idx_row], word_v.at[pl.ds(p * C, C)],
                         sem_w.at[p])
        pltpu.async_copy(ft_hbm.at[idx_row], flair_v.at[pl.ds(p * C, C)],
                         sem_f.at[p])

    def wait_gathers(j, p):
        idx_row = idx_v.at[pl.ds(j * C, C)]
        pltpu.make_async_copy(wt_hbm.at[idx_row],
                              word_v.at[pl.ds(p * C, C)], sem_w.at[p]).wait()
        pltpu.make_async_copy(ft_hbm.at[idx_row],
                              flair_v.at[pl.ds(p * C, C)], sem_f.at[p]).wait()

    def out_slice(j):
        return out_hbm.at[pl.ds((wid * TPW + j * C) * D, C * D)]

    start_gathers(0, 0)

    def chunk(j, carry):
        p = j % 2

        @pl.when(j + 1 < NCH)
        def _():
            start_gathers(j + 1, 1 - p)

        wait_gathers(j, p)

        @pl.when(j >= 2)
        def _():
            pltpu.make_async_copy(comb_v.at[pl.ds(p * C * D, C * D)],
                                  out_slice(j - 2), sem_o.at[p]).wait()

        def assemble(t, carry2):
            cbase = p * C * D + t * D
            row = p * C + t
            row_vec = jnp.full((L,), row, jnp.int32)
            for k in range(DW // L):
                comb_v[pl.ds(cbase + k * L, L)] = word_v[row, pl.ds(k * L, L)]
            v = plsc.load_gather(word_v, [row_vec, (DW - L) + lanes])
            comb_v[pl.ds(cbase + DW - L, L)] = v
            for k in range(DF // L):
                comb_v[pl.ds(cbase + DW + k * L, L)] = \
                    flair_v[row, pl.ds(k * L, L)]
            return carry2

        lax.fori_loop(0, C, assemble, 0)
        pltpu.async_copy(comb_v.at[pl.ds(p * C * D, C * D)], out_slice(j),
                         sem_o.at[p])
        return carry

    lax.fori_loop(0, NCH, chunk, 0)

    pltpu.make_async_copy(comb_v.at[pl.ds(((NCH - 2) % 2) * C * D, C * D)],
                          out_slice(NCH - 2), sem_o.at[(NCH - 2) % 2]).wait()
    pltpu.make_async_copy(comb_v.at[pl.ds(((NCH - 1) % 2) * C * D, C * D)],
                          out_slice(NCH - 1), sem_o.at[(NCH - 1) % 2]).wait()


def kernel(x, word_table, flair_table):
    x1 = x.reshape(T)
    nv = word_table.shape[0]
    rb = 2000

    def _pad_body(w_ref, o_ref):
        o_ref[...] = jnp.concatenate(
            [w_ref[...], jnp.zeros((rb, DWP - DW), jnp.float32)], axis=1)

    wt_p = pl.pallas_call(
        _pad_body,
        grid=(nv // rb,),
        in_specs=[pl.BlockSpec((rb, DW), lambda i: (i, 0))],
        out_specs=pl.BlockSpec((rb, DWP), lambda i: (i, 0)),
        out_shape=jax.ShapeDtypeStruct((nv, DWP), jnp.float32),
    )(word_table)
    mesh = plsc.VectorSubcoreMesh(core_axis_name="c", subcore_axis_name="s",
                                  num_cores=NC, num_subcores=NS)
    out = pl.kernel(
        _sc_body,
        out_type=jax.ShapeDtypeStruct((T * D,), jnp.float32),
        mesh=mesh,
        compiler_params=pltpu.CompilerParams(use_tc_tiling_on_sc=True,
                                             needs_layout_passes=False),
        scratch_types=[
            pltpu.VMEM((TPW,), jnp.int32),
            pltpu.VMEM((2 * C, DWP), jnp.float32),
            pltpu.VMEM((2 * C, DF), jnp.float32),
            pltpu.VMEM((2 * C * D,), jnp.float32),
            pltpu.SemaphoreType.DMA((2,)),
            pltpu.SemaphoreType.DMA((2,)),
            pltpu.SemaphoreType.DMA((2,)),
        ],
    )(x1, wt_p, flair_table)
    return out


# speedup vs baseline: 1.2296x; 1.2296x over previous
import jax
import jax.numpy as jnp
from jax import lax
from jax.experimental import pallas as pl
from jax.experimental.pallas import tpu as pltpu
from jax.experimental.pallas import tpu_sc as plsc

NC, NS, L = 2, 16, 16
NW = NC * NS
T = 1024 * 50
TPW = T // NW
C = 80
NCH = TPW // C
DW, DF = 100, 256
D = DW + DF
DWP = 128


def _sc_body(x_hbm, wt_hbm, ft_hbm, out_hbm, idx_v, word_v, flair_v, comb_v,
             sem_w, sem_f, sem_o):
    wid = lax.axis_index("s") * NC + lax.axis_index("c")
    pltpu.sync_copy(x_hbm.at[pl.ds(wid * TPW, TPW)], idx_v)

    lanes = lax.iota(jnp.int32, L)

    def start_gathers(j, p):
        idx_row = idx_v.at[pl.ds(j * C, C)]
        pltpu.async_copy(wt_hbm.at[idx_row], word_v.at[pl.ds(p * C, C)],
                         sem_w.at[p])
        pltpu.async_copy(ft_hbm.at[idx_row], flair_v.at[pl.ds(p * C, C)],
                         sem_f.at[p])

    def wait_gathers(j, p):
        idx_row = idx_v.at[pl.ds(j * C, C)]
        pltpu.make_async_copy(wt_hbm.at[idx_row],
                              word_v.at[pl.ds(p * C, C)], sem_w.at[p]).wait()
        pltpu.make_async_copy(ft_hbm.at[idx_row],
                              flair_v.at[pl.ds(p * C, C)], sem_f.at[p]).wait()

    def out_slice(j):
        return out_hbm.at[pl.ds((wid * TPW + j * C) * D, C * D)]

    start_gathers(0, 0)

    def chunk(j, carry):
        p = j % 2

        @pl.when(j + 1 < NCH)
        def _():
            start_gathers(j + 1, 1 - p)

        wait_gathers(j, p)

        @pl.when(j >= 2)
        def _():
            pltpu.make_async_copy(comb_v.at[pl.ds(p * C * D, C * D)],
                                  out_slice(j - 2), sem_o.at[p]).wait()

        def assemble(t, carry2):
            cbase = p * C * D + t * D
            row = p * C + t
            row_vec = jnp.full((L,), row, jnp.int32)
            for k in range(DW // L):
                comb_v[pl.ds(cbase + k * L, L)] = word_v[row, pl.ds(k * L, L)]
            v = plsc.load_gather(word_v, [row_vec, (DW - L) + lanes])
            comb_v[pl.ds(cbase + DW - L, L)] = v
            for k in range(DF // L):
                comb_v[pl.ds(cbase + DW + k * L, L)] = \
                    flair_v[row, pl.ds(k * L, L)]
            return carry2

        lax.fori_loop(0, C, assemble, 0)
        pltpu.async_copy(comb_v.at[pl.ds(p * C * D, C * D)], out_slice(j),
                         sem_o.at[p])
        return carry

    lax.fori_loop(0, NCH, chunk, 0)

    pltpu.make_async_copy(comb_v.at[pl.ds(((NCH - 2) % 2) * C * D, C * D)],
                          out_slice(NCH - 2), sem_o.at[(NCH - 2) % 2]).wait()
    pltpu.make_async_copy(comb_v.at[pl.ds(((NCH - 1) % 2) * C * D, C * D)],
                          out_slice(NCH - 1), sem_o.at[(NCH - 1) % 2]).wait()


def kernel(x, word_table, flair_table):
    x1 = x.reshape(T)
    pad_eye = jnp.eye(DW, DWP, dtype=jnp.float32)
    wt_p = lax.dot_general(word_table, pad_eye, (((1,), (0,)), ((), ())),
                           precision=lax.Precision.HIGHEST)
    mesh = plsc.VectorSubcoreMesh(core_axis_name="c", subcore_axis_name="s",
                                  num_cores=NC, num_subcores=NS)
    out = pl.kernel(
        _sc_body,
        out_type=jax.ShapeDtypeStruct((T * D,), jnp.float32),
        mesh=mesh,
        compiler_params=pltpu.CompilerParams(use_tc_tiling_on_sc=True,
                                             needs_layout_passes=False),
        scratch_types=[
            pltpu.VMEM((TPW,), jnp.int32),
            pltpu.VMEM((2 * C, DWP), jnp.float32),
            pltpu.VMEM((2 * C, DF), jnp.float32),
            pltpu.VMEM((2 * C * D,), jnp.float32),
            pltpu.SemaphoreType.DMA((2,)),
            pltpu.SemaphoreType.DMA((2,)),
            pltpu.SemaphoreType.DMA((2,)),
        ],
    )(x1, wt_p, flair_table)
    return out


# speedup vs baseline: 1.2397x; 1.0082x over previous
import jax
import jax.numpy as jnp
from jax import lax
from jax.experimental import pallas as pl
from jax.experimental.pallas import tpu as pltpu
from jax.experimental.pallas import tpu_sc as plsc

NC, NS, L = 2, 16, 16
NW = NC * NS
T = 1024 * 50
TPW = T // NW
C = 80
NCH = TPW // C
DW, DF = 100, 256
D = DW + DF
DWP = 128


def _sc_body(x_hbm, wt_hbm, ft_hbm, out_hbm, idx_v, word_v, flair_v, comb_v,
             sem_w, sem_f, sem_o):
    wid = lax.axis_index("s") * NC + lax.axis_index("c")
    pltpu.sync_copy(x_hbm.at[pl.ds(wid * TPW, TPW)], idx_v)

    lanes = lax.iota(jnp.int32, L)

    def start_gathers(j, p):
        idx_row = idx_v.at[pl.ds(j * C, C)]
        pltpu.async_copy(wt_hbm.at[idx_row], word_v.at[pl.ds(p * C, C)],
                         sem_w.at[p])
        pltpu.async_copy(ft_hbm.at[idx_row], flair_v.at[pl.ds(p * C, C)],
                         sem_f.at[p])

    def wait_gathers(j, p):
        idx_row = idx_v.at[pl.ds(j * C, C)]
        pltpu.make_async_copy(wt_hbm.at[idx_row],
                              word_v.at[pl.ds(p * C, C)], sem_w.at[p]).wait()
        pltpu.make_async_copy(ft_hbm.at[idx_row],
                              flair_v.at[pl.ds(p * C, C)], sem_f.at[p]).wait()

    def out_slice(j):
        return out_hbm.at[pl.ds((wid * TPW + j * C) * D, C * D)]

    start_gathers(0, 0)

    def chunk(j, carry):
        p = j % 2
        wait_gathers(j, p)

        @pl.when(j + 1 < NCH)
        def _():
            start_gathers(j + 1, 1 - p)

        @pl.when(j >= 2)
        def _():
            pltpu.make_async_copy(comb_v.at[pl.ds(p * C * D, C * D)],
                                  out_slice(j - 2), sem_o.at[p]).wait()

        def assemble(t, carry2):
            cbase = p * C * D + t * D
            row = p * C + t
            row_vec = jnp.full((L,), row, jnp.int32)
            for k in range(DW // L):
                comb_v[pl.ds(cbase + k * L, L)] = word_v[row, pl.ds(k * L, L)]
            v = plsc.load_gather(word_v, [row_vec, (DW - L) + lanes])
            comb_v[pl.ds(cbase + DW - L, L)] = v
            for k in range(DF // L):
                comb_v[pl.ds(cbase + DW + k * L, L)] = \
                    flair_v[row, pl.ds(k * L, L)]
            return carry2

        lax.fori_loop(0, C, assemble, 0)
        pltpu.async_copy(comb_v.at[pl.ds(p * C * D, C * D)], out_slice(j),
                         sem_o.at[p])
        return carry

    lax.fori_loop(0, NCH, chunk, 0)

    pltpu.make_async_copy(comb_v.at[pl.ds(((NCH - 2) % 2) * C * D, C * D)],
                          out_slice(NCH - 2), sem_o.at[(NCH - 2) % 2]).wait()
    pltpu.make_async_copy(comb_v.at[pl.ds(((NCH - 1) % 2) * C * D, C * D)],
                          out_slice(NCH - 1), sem_o.at[(NCH - 1) % 2]).wait()


def kernel(x, word_table, flair_table):
    x1 = x.reshape(T)
    pad_eye = jnp.eye(DW, DWP, dtype=jnp.float32)
    wt_p = lax.dot_general(word_table, pad_eye, (((1,), (0,)), ((), ())),
                           precision=lax.Precision.HIGHEST)
    mesh = plsc.VectorSubcoreMesh(core_axis_name="c", subcore_axis_name="s",
                                  num_cores=NC, num_subcores=NS)
    out = pl.kernel(
        _sc_body,
        out_type=jax.ShapeDtypeStruct((T * D,), jnp.float32),
        mesh=mesh,
        compiler_params=pltpu.CompilerParams(use_tc_tiling_on_sc=True,
                                             needs_layout_passes=False),
        scratch_types=[
            pltpu.VMEM((TPW,), jnp.int32),
            pltpu.VMEM((2 * C, DWP), jnp.float32),
            pltpu.VMEM((2 * C, DF), jnp.float32),
            pltpu.VMEM((2 * C * D,), jnp.float32),
            pltpu.SemaphoreType.DMA((2,)),
            pltpu.SemaphoreType.DMA((2,)),
            pltpu.SemaphoreType.DMA((2,)),
        ],
    )(x1, wt_p, flair_table)
    return out
